# baseline (speedup 1.0000x reference)
import functools

import jax
import jax.numpy as jnp
from jax import lax
from jax.experimental import pallas as pl
from jax.experimental.pallas import tpu as pltpu
from jax.experimental.pallas import tpu_sc as plsc

_N = 10000
_NP = 10240
_E = 320000
_F = 128
_NCORES = 2
_NSUB = 16
_NTILES = _NCORES * _NSUB
_CHUNK_DEG = 128
_CPT_DEG = 80
_CHUNK = 64
_CPT = 160
_CPT_A = 112
_CPT_B = 208
_NE_TILE = _CPT * _CHUNK
_E_PAD = _NE_TILE * _NTILES
_NROW = _E_PAD // 128
_ROWS_PER_TILE = _NP // _NSUB

_mesh = plsc.VectorSubcoreMesh(core_axis_name="c", subcore_axis_name="s")


def _sc_deg_body(dst_hbm, ew_hbm, out_hbm, didx_all, ew_all, buf_v,
                 sem0, sem1, deg_sh):
    cid = lax.axis_index("c")
    sid = lax.axis_index("s")
    tile = cid * _NSUB + sid
    r0 = sid * _ROWS_PER_TILE

    pltpu.sync_copy(dst_hbm.at[tile], didx_all)
    pltpu.sync_copy(ew_hbm.at[tile], ew_all)

    def zfill(j, carry):
        buf_v[pl.ds(j * 16, 16)] = jnp.zeros((16,), jnp.float32)
        return carry

    lax.fori_loop(0, _ROWS_PER_TILE // 16, zfill, 0)
    pltpu.sync_copy(buf_v, deg_sh.at[pl.ds(r0, _ROWS_PER_TILE)])
    plsc.subcore_barrier()

    sems = (sem0, sem1)

    def pair(p, carry):
        for b in range(2):
            cc = 2 * p + b

            @pl.when(p > 0)
            def _wait():
                pltpu.make_async_copy(
                    ew_all.at[0], deg_sh.at[didx_all.at[0]], sems[b]).wait()

            pltpu.async_copy(
                ew_all.at[cc], deg_sh.at[didx_all.at[cc]], sems[b], add=True)
        return carry

    lax.fori_loop(0, _CPT_DEG // 2, pair, 0)
    for b in range(2):
        pltpu.make_async_copy(
            ew_all.at[0], deg_sh.at[didx_all.at[0]], sems[b]).wait()
    plsc.subcore_barrier()
    pltpu.sync_copy(deg_sh.at[pl.ds(r0, _ROWS_PER_TILE)], buf_v)
    pltpu.sync_copy(buf_v, out_hbm.at[cid, pl.ds(r0, _ROWS_PER_TILE)])


_sc_deg = pl.kernel(
    _sc_deg_body,
    out_type=jax.ShapeDtypeStruct((_NCORES, _NP), jnp.float32),
    mesh=_mesh,
    scratch_types=[
        pltpu.VMEM((_CPT_DEG, _CHUNK_DEG), jnp.int32),
        pltpu.VMEM((_CPT_DEG, _CHUNK_DEG), jnp.float32),
        pltpu.VMEM((_ROWS_PER_TILE,), jnp.float32),
        pltpu.SemaphoreType.DMA,
        pltpu.SemaphoreType.DMA,
        pltpu.VMEM_SHARED((_NP,), jnp.float32),
    ],
)


def _sc_main_body(pk_hbm, ew_hbm, xs_hbm, out_hbm,
                  pk_all, ew_all, si0, si1, si2, si3, di0, di1, di2, di3,
                  rg0, rg1, rs,
                  gsem0, gsem1, ssem, acc_sh):
    cid = lax.axis_index("c")
    sid = lax.axis_index("s")
    r0 = sid * _ROWS_PER_TILE

    cpt = jnp.where(cid == 0, _CPT_A, _CPT_B)
    rowbase = jnp.where(
        cid == 0, sid * (_CPT_A // 2),
        _NSUB * (_CPT_A // 2) + sid * (_CPT_B // 2))

    @pl.when(cid == 0)
    def _prefetch_a():
        pltpu.sync_copy(pk_hbm.at[pl.ds(rowbase, _CPT_A // 2)],
                        pk_all.at[pl.ds(0, _CPT_A // 2)])

    @pl.when(cid == 1)
    def _prefetch_b():
        pltpu.sync_copy(pk_hbm.at[pl.ds(rowbase, _CPT_B // 2)], pk_all)
    pltpu.sync_copy(ew_hbm.at[pl.ds(rowbase, _CPT // 4)],
                    ew_all)

    sib = (si0, si1, si2, si3)
    dib = (di0, di1, di2, di3)
    rg = (rg0, rg1)
    gsems = (gsem0, gsem1)

    mask = jnp.full((16,), 0xFFFF, jnp.int32)

    def unpack(cc, j, row, colbase):
        for v in range(_CHUNK // 16):
            pk = pk_all[row, pl.ds(colbase + v * 16, 16)]
            sib[j][pl.ds(v * 16, 16)] = jnp.bitwise_and(pk, mask)
            dib[j][pl.ds(v * 16, 16)] = lax.shift_right_logical(pk, 16)

    def zrow(j, carry):
        for f in range(_F // 16):
            rs[j, pl.ds(f * 16, 16)] = jnp.zeros((16,), jnp.float32)
        return carry

    lax.fori_loop(0, _CHUNK, zrow, 0)
    for k in range(_ROWS_PER_TILE // _CHUNK):
        pltpu.sync_copy(rs, acc_sh.at[pl.ds(r0 + k * _CHUNK, _CHUNK)])

    for b in range(2):
        unpack(b, b, 0, b * _CHUNK)
        pltpu.async_copy(xs_hbm.at[sib[b]], rg[b], gsems[b])
    plsc.subcore_barrier()

    def quad(q, carry):
        @pl.when(jnp.logical_and(q == _CPT // 8, cid == 0))
        def _reload_ew_a():
            pltpu.sync_copy(
                ew_hbm.at[pl.ds(rowbase + _CPT // 4, _CPT_A // 2 - _CPT // 4)],
                ew_all.at[pl.ds(0, _CPT_A // 2 - _CPT // 4)])

        @pl.when(jnp.logical_and(q == _CPT // 8, cid == 1))
        def _reload_ew_b():
            pltpu.sync_copy(
                ew_hbm.at[pl.ds(rowbase + _CPT // 4, _CPT // 4)], ew_all)

        @pl.when(jnp.logical_and(q == _CPT // 4, cid == 1))
        def _reload_ew_b2():
            pltpu.sync_copy(
                ew_hbm.at[pl.ds(rowbase + _CPT // 2, _CPT_B // 2 - _CPT // 2)],
                ew_all.at[pl.ds(0, _CPT_B // 2 - _CPT // 2)])

        for j in range(4):
            b = j % 2
            cc = 4 * q + j
            pltpu.make_async_copy(
                xs_hbm.at[sib[0]], rg[b], gsems[b]).wait()

            def _wait_scatter():
                pltpu.make_async_copy(
                    rs, acc_sh.at[dib[0]], ssem).wait()

            if j == 0:
                pl.when(q > 0)(_wait_scatter)
            else:
                _wait_scatter()

            row = 2 * q + (j // 2)
            row_ew = lax.rem(row, _CPT // 4)
            colbase = (j % 2) * _CHUNK

            def scale(g, c2):
                ew16 = ew_all[row_ew, pl.ds(colbase + g * 16, 16)]
                for l in range(16):
                    w = jnp.full((16,), ew16[l], jnp.float32)
                    i = g * 16 + l
                    for f in range(_F // 16):
                        rs[i, pl.ds(f * 16, 16)] = (
                            rg[b][i, pl.ds(f * 16, 16)] * w)
                return c2

            lax.fori_loop(0, _CHUNK // 16, scale, 0)
            pltpu.async_copy(rs, acc_sh.at[dib[j]], ssem, add=True)

            def _next():
                unpack(cc + 2, (j + 2) % 4,
                       2 * q + 1 + (j // 2), ((j + 2) % 2) * _CHUNK)
                pltpu.async_copy(
                    xs_hbm.at[sib[(j + 2) % 4]], rg[b], gsems[b])

            if j < 2:
                _next()
            else:
                pl.when(q < cpt // 4 - 1)(_next)
        return carry

    lax.fori_loop(0, cpt // 4, quad, 0)
    pltpu.make_async_copy(rs, acc_sh.at[dib[0]], ssem).wait()
    plsc.subcore_barrier()
    for k in range(_ROWS_PER_TILE // _CHUNK):
        rr = r0 + k * _CHUNK
        pltpu.sync_copy(acc_sh.at[pl.ds(rr, _CHUNK)], rs)
        pltpu.sync_copy(rs, out_hbm.at[cid, pl.ds(rr, _CHUNK)])


_sc_main = pl.kernel(
    _sc_main_body,
    out_type=jax.ShapeDtypeStruct((_NCORES, _NP, _F), jnp.float32),
    mesh=_mesh,
    scratch_types=(
        [pltpu.VMEM((_CPT_B // 2, 2 * _CHUNK), jnp.int32)]
        + [pltpu.VMEM((_CPT // 4, 2 * _CHUNK), jnp.float32)]
        + [pltpu.VMEM((_CHUNK,), jnp.int32) for _ in range(8)]
        + [pltpu.VMEM((_CHUNK, _F), jnp.float32) for _ in range(3)]
        + [pltpu.SemaphoreType.DMA for _ in range(3)]
        + [pltpu.VMEM_SHARED((_NP, _F), jnp.float32)]
    ),
)


def _tc_mid_body(x_ref, w_ref, degp_ref, xs_ref):
    deg = degp_ref[0:1, :] + degp_ref[1:2, :] + 1.0
    dinv = lax.rsqrt(deg)
    dcol = dinv.reshape(_NP, 1)[:_N]
    xw = jnp.dot(x_ref[...], w_ref[...], preferred_element_type=jnp.float32)
    xs_ref[...] = xw * dcol


def _tc_post_body(accp_ref, xs_ref, degp_ref, bzh_ref, wlz_ref, blz_ref,
                  wlh_ref, blh_ref, wo_ref, bo_ref, y_ref, h_ref):
    deg = degp_ref[0:1, :] + degp_ref[1:2, :] + 1.0
    dinv = lax.rsqrt(deg)
    dcol = dinv.reshape(_NP, 1)[:_N]
    acc = accp_ref[0, :_N, :] + accp_ref[1, :_N, :] + xs_ref[...]
    og = acc * dcol + bzh_ref[...]
    cz = og[:, :64]
    ch = og[:, 64:]
    z = jax.nn.sigmoid(
        jnp.dot(cz, wlz_ref[...], preferred_element_type=jnp.float32)
        + blz_ref[...])
    ht = jnp.tanh(
        jnp.dot(ch, wlh_ref[...], preferred_element_type=jnp.float32)
        + blh_ref[...])
    h = (1.0 - z) * ht
    h_ref[...] = h
    y_ref[...] = (
        jnp.dot(jnp.maximum(h, 0.0), wo_ref[...],
                preferred_element_type=jnp.float32)
        + bo_ref[...])


def kernel(x, edge_index, edge_weight, Wz, bz, Wr, br, Wh, bh,
           Wlz, blz, Wlr, blr, Wlh, blh, Wo, bo):
    f_out = Wz.shape[1]
    src = edge_index[0]
    dst = edge_index[1]
    pad = _E_PAD - _E
    src_p = jnp.concatenate([src, jnp.zeros((pad,), src.dtype)])
    dst_p = jnp.concatenate([dst, jnp.zeros((pad,), dst.dtype)])
    ew_p = jnp.concatenate([edge_weight, jnp.zeros((pad,), edge_weight.dtype)])
    packed = jnp.bitwise_or(src_p, lax.shift_left(dst_p, 16))
    pk3 = packed.reshape(_NROW, 2 * _CHUNK)
    ewb3 = ew_p.reshape(_NROW, 2 * _CHUNK)
    dst_deg = dst_p.reshape(_NTILES, _CPT_DEG, _CHUNK_DEG)
    ew_deg = ew_p.reshape(_NTILES, _CPT_DEG, _CHUNK_DEG)
    wzh = jnp.concatenate([Wz, Wh], axis=1)
    bzh = jnp.concatenate([bz, bh]).reshape(1, _F)

    degp = _sc_deg(dst_deg, ew_deg)

    xs = pl.pallas_call(
        _tc_mid_body,
        out_shape=jax.ShapeDtypeStruct((_N, _F), jnp.float32),
    )(x, wzh, degp)

    accp = _sc_main(pk3, ewb3, xs)

    y, h = pl.pallas_call(
        _tc_post_body,
        out_shape=(
            jax.ShapeDtypeStruct((_N, 1), jnp.float32),
            jax.ShapeDtypeStruct((_N, f_out), jnp.float32),
        ),
    )(accp, xs, degp, bzh, Wlz[:f_out], blz.reshape(1, f_out),
      Wlh[:f_out], blh.reshape(1, f_out), Wo, bo.reshape(1, 1))

    return (y, h)


# speedup vs baseline: 1.1792x; 1.1792x over previous
import functools

import jax
import jax.numpy as jnp
from jax import lax
from jax.experimental import pallas as pl
from jax.experimental.pallas import tpu as pltpu
from jax.experimental.pallas import tpu_sc as plsc

_N = 10000
_NP = 10240
_E = 320000
_F = 128
_NCORES = 2
_NSUB = 16
_NTILES = _NCORES * _NSUB
_CHUNK_DEG = 128
_CPT_DEG = 80
_CHUNK = 64
_CPT = 160
_CPT_A = 208
_CPT_B = 112
_NE_TILE = _CPT * _CHUNK
_E_PAD = _NE_TILE * _NTILES
_NROW = _E_PAD // 128
_ROWS_PER_TILE = _NP // _NSUB

_mesh = plsc.VectorSubcoreMesh(core_axis_name="c", subcore_axis_name="s")


def _sc_deg_body(dst_hbm, ew_hbm, out_hbm, didx_all, ew_all, buf_v,
                 sem0, sem1, deg_sh):
    cid = lax.axis_index("c")
    sid = lax.axis_index("s")
    tile = cid * _NSUB + sid
    r0 = sid * _ROWS_PER_TILE

    pltpu.sync_copy(dst_hbm.at[tile], didx_all)
    pltpu.sync_copy(ew_hbm.at[tile], ew_all)

    def zfill(j, carry):
        buf_v[pl.ds(j * 16, 16)] = jnp.zeros((16,), jnp.float32)
        return carry

    lax.fori_loop(0, _ROWS_PER_TILE // 16, zfill, 0)
    pltpu.sync_copy(buf_v, deg_sh.at[pl.ds(r0, _ROWS_PER_TILE)])
    plsc.subcore_barrier()

    sems = (sem0, sem1)

    def pair(p, carry):
        for b in range(2):
            cc = 2 * p + b

            @pl.when(p > 0)
            def _wait():
                pltpu.make_async_copy(
                    ew_all.at[0], deg_sh.at[didx_all.at[0]], sems[b]).wait()

            pltpu.async_copy(
                ew_all.at[cc], deg_sh.at[didx_all.at[cc]], sems[b], add=True)
        return carry

    lax.fori_loop(0, _CPT_DEG // 2, pair, 0)
    for b in range(2):
        pltpu.make_async_copy(
            ew_all.at[0], deg_sh.at[didx_all.at[0]], sems[b]).wait()
    plsc.subcore_barrier()
    pltpu.sync_copy(deg_sh.at[pl.ds(r0, _ROWS_PER_TILE)], buf_v)
    pltpu.sync_copy(buf_v, out_hbm.at[cid, pl.ds(r0, _ROWS_PER_TILE)])


_sc_deg = pl.kernel(
    _sc_deg_body,
    out_type=jax.ShapeDtypeStruct((_NCORES, _NP), jnp.float32),
    mesh=_mesh,
    scratch_types=[
        pltpu.VMEM((_CPT_DEG, _CHUNK_DEG), jnp.int32),
        pltpu.VMEM((_CPT_DEG, _CHUNK_DEG), jnp.float32),
        pltpu.VMEM((_ROWS_PER_TILE,), jnp.float32),
        pltpu.SemaphoreType.DMA,
        pltpu.SemaphoreType.DMA,
        pltpu.VMEM_SHARED((_NP,), jnp.float32),
    ],
)


def _sc_main_body(pk_hbm, ew_hbm, xs_hbm, out_hbm,
                  pk_all, ew_all, si0, si1, si2, si3, di0, di1, di2, di3,
                  rg0, rg1, rs,
                  gsem0, gsem1, ssem, acc_sh):
    cid = lax.axis_index("c")
    sid = lax.axis_index("s")
    r0 = sid * _ROWS_PER_TILE

    cpt = jnp.where(cid == 0, _CPT_A, _CPT_B)
    rowbase = jnp.where(
        cid == 0, sid * (_CPT_A // 2),
        _NSUB * (_CPT_A // 2) + sid * (_CPT_B // 2))

    @pl.when(cid == 0)
    def _prefetch_a():
        pltpu.sync_copy(pk_hbm.at[pl.ds(rowbase, _CPT_A // 2)], pk_all)

    @pl.when(cid == 1)
    def _prefetch_b():
        pltpu.sync_copy(pk_hbm.at[pl.ds(rowbase, _CPT_B // 2)],
                        pk_all.at[pl.ds(0, _CPT_B // 2)])
    pltpu.sync_copy(ew_hbm.at[pl.ds(rowbase, _CPT // 4)],
                    ew_all)

    sib = (si0, si1, si2, si3)
    dib = (di0, di1, di2, di3)
    rg = (rg0, rg1)
    gsems = (gsem0, gsem1)

    mask = jnp.full((16,), 0xFFFF, jnp.int32)

    def unpack(cc, j, row, colbase):
        for v in range(_CHUNK // 16):
            pk = pk_all[row, pl.ds(colbase + v * 16, 16)]
            sib[j][pl.ds(v * 16, 16)] = jnp.bitwise_and(pk, mask)
            dib[j][pl.ds(v * 16, 16)] = lax.shift_right_logical(pk, 16)

    def zrow(j, carry):
        for f in range(_F // 16):
            rs[j, pl.ds(f * 16, 16)] = jnp.zeros((16,), jnp.float32)
        return carry

    lax.fori_loop(0, _CHUNK, zrow, 0)
    for k in range(_ROWS_PER_TILE // _CHUNK):
        pltpu.sync_copy(rs, acc_sh.at[pl.ds(r0 + k * _CHUNK, _CHUNK)])

    for b in range(2):
        unpack(b, b, 0, b * _CHUNK)
        pltpu.async_copy(xs_hbm.at[sib[b]], rg[b], gsems[b])
    plsc.subcore_barrier()

    def quad(q, carry):
        @pl.when(jnp.logical_and(q == _CPT // 8, cid == 1))
        def _reload_small():
            pltpu.sync_copy(
                ew_hbm.at[pl.ds(rowbase + _CPT // 4, _CPT_B // 2 - _CPT // 4)],
                ew_all.at[pl.ds(0, _CPT_B // 2 - _CPT // 4)])

        @pl.when(jnp.logical_and(q == _CPT // 8, cid == 0))
        def _reload_big1():
            pltpu.sync_copy(
                ew_hbm.at[pl.ds(rowbase + _CPT // 4, _CPT // 4)], ew_all)

        @pl.when(jnp.logical_and(q == _CPT // 4, cid == 0))
        def _reload_big2():
            pltpu.sync_copy(
                ew_hbm.at[pl.ds(rowbase + _CPT // 2, _CPT_A // 2 - _CPT // 2)],
                ew_all.at[pl.ds(0, _CPT_A // 2 - _CPT // 2)])

        for j in range(4):
            b = j % 2
            cc = 4 * q + j
            pltpu.make_async_copy(
                xs_hbm.at[sib[0]], rg[b], gsems[b]).wait()

            def _wait_scatter():
                pltpu.make_async_copy(
                    rs, acc_sh.at[dib[0]], ssem).wait()

            if j == 0:
                pl.when(q > 0)(_wait_scatter)
            else:
                _wait_scatter()

            row = 2 * q + (j // 2)
            row_ew = lax.rem(row, _CPT // 4)
            colbase = (j % 2) * _CHUNK

            def scale(g, c2):
                ew16 = ew_all[row_ew, pl.ds(colbase + g * 16, 16)]
                for l in range(16):
                    w = jnp.full((16,), ew16[l], jnp.float32)
                    i = g * 16 + l
                    for f in range(_F // 16):
                        rs[i, pl.ds(f * 16, 16)] = (
                            rg[b][i, pl.ds(f * 16, 16)] * w)
                return c2

            lax.fori_loop(0, _CHUNK // 16, scale, 0)
            pltpu.async_copy(rs, acc_sh.at[dib[j]], ssem, add=True)

            def _next():
                unpack(cc + 2, (j + 2) % 4,
                       2 * q + 1 + (j // 2), ((j + 2) % 2) * _CHUNK)
                pltpu.async_copy(
                    xs_hbm.at[sib[(j + 2) % 4]], rg[b], gsems[b])

            if j < 2:
                _next()
            else:
                pl.when(q < cpt // 4 - 1)(_next)
        return carry

    lax.fori_loop(0, cpt // 4, quad, 0)
    pltpu.make_async_copy(rs, acc_sh.at[dib[0]], ssem).wait()
    plsc.subcore_barrier()
    for k in range(_ROWS_PER_TILE // _CHUNK):
        rr = r0 + k * _CHUNK
        pltpu.sync_copy(acc_sh.at[pl.ds(rr, _CHUNK)], rs)
        pltpu.sync_copy(rs, out_hbm.at[cid, pl.ds(rr, _CHUNK)])


_sc_main = pl.kernel(
    _sc_main_body,
    out_type=jax.ShapeDtypeStruct((_NCORES, _NP, _F), jnp.float32),
    mesh=_mesh,
    scratch_types=(
        [pltpu.VMEM((_CPT_A // 2, 2 * _CHUNK), jnp.int32)]
        + [pltpu.VMEM((_CPT // 4, 2 * _CHUNK), jnp.float32)]
        + [pltpu.VMEM((_CHUNK,), jnp.int32) for _ in range(8)]
        + [pltpu.VMEM((_CHUNK, _F), jnp.float32) for _ in range(3)]
        + [pltpu.SemaphoreType.DMA for _ in range(3)]
        + [pltpu.VMEM_SHARED((_NP, _F), jnp.float32)]
    ),
)


def _tc_mid_body(x_ref, w_ref, degp_ref, xs_ref):
    deg = degp_ref[0:1, :] + degp_ref[1:2, :] + 1.0
    dinv = lax.rsqrt(deg)
    dcol = dinv.reshape(_NP, 1)[:_N]
    xw = jnp.dot(x_ref[...], w_ref[...], preferred_element_type=jnp.float32)
    xs_ref[...] = xw * dcol


def _tc_post_body(accp_ref, xs_ref, degp_ref, bzh_ref, wlz_ref, blz_ref,
                  wlh_ref, blh_ref, wo_ref, bo_ref, y_ref, h_ref):
    deg = degp_ref[0:1, :] + degp_ref[1:2, :] + 1.0
    dinv = lax.rsqrt(deg)
    dcol = dinv.reshape(_NP, 1)[:_N]
    acc = accp_ref[0, :_N, :] + accp_ref[1, :_N, :] + xs_ref[...]
    og = acc * dcol + bzh_ref[...]
    cz = og[:, :64]
    ch = og[:, 64:]
    z = jax.nn.sigmoid(
        jnp.dot(cz, wlz_ref[...], preferred_element_type=jnp.float32)
        + blz_ref[...])
    ht = jnp.tanh(
        jnp.dot(ch, wlh_ref[...], preferred_element_type=jnp.float32)
        + blh_ref[...])
    h = (1.0 - z) * ht
    h_ref[...] = h
    y_ref[...] = (
        jnp.dot(jnp.maximum(h, 0.0), wo_ref[...],
                preferred_element_type=jnp.float32)
        + bo_ref[...])


def kernel(x, edge_index, edge_weight, Wz, bz, Wr, br, Wh, bh,
           Wlz, blz, Wlr, blr, Wlh, blh, Wo, bo):
    f_out = Wz.shape[1]
    src = edge_index[0]
    dst = edge_index[1]
    pad = _E_PAD - _E
    src_p = jnp.concatenate([src, jnp.zeros((pad,), src.dtype)])
    dst_p = jnp.concatenate([dst, jnp.zeros((pad,), dst.dtype)])
    ew_p = jnp.concatenate([edge_weight, jnp.zeros((pad,), edge_weight.dtype)])
    packed = jnp.bitwise_or(src_p, lax.shift_left(dst_p, 16))
    pk3 = packed.reshape(_NROW, 2 * _CHUNK)
    ewb3 = ew_p.reshape(_NROW, 2 * _CHUNK)
    dst_deg = dst_p.reshape(_NTILES, _CPT_DEG, _CHUNK_DEG)
    ew_deg = ew_p.reshape(_NTILES, _CPT_DEG, _CHUNK_DEG)
    wzh = jnp.concatenate([Wz, Wh], axis=1)
    bzh = jnp.concatenate([bz, bh]).reshape(1, _F)

    degp = _sc_deg(dst_deg, ew_deg)

    xs = pl.pallas_call(
        _tc_mid_body,
        out_shape=jax.ShapeDtypeStruct((_N, _F), jnp.float32),
    )(x, wzh, degp)

    accp = _sc_main(pk3, ewb3, xs)

    y, h = pl.pallas_call(
        _tc_post_body,
        out_shape=(
            jax.ShapeDtypeStruct((_N, 1), jnp.float32),
            jax.ShapeDtypeStruct((_N, f_out), jnp.float32),
        ),
    )(accp, xs, degp, bzh, Wlz[:f_out], blz.reshape(1, f_out),
      Wlh[:f_out], blh.reshape(1, f_out), Wo, bo.reshape(1, 1))

    return (y, h)
